# baseline (speedup 1.0000x reference)
import jax
import jax.numpy as jnp
from jax.experimental import pallas as pl

DIM = 256
N_CODES = 8192
CHUNK = 2048
BR = 512


def _assign_kernel(x_ref, xn_ref, e2_ref, en_ref, ids_ref, out_ref):
    xb = x_ref[...]
    x_norm = xn_ref[0, 0, :][:, None]
    en = en_ref[...]
    e2 = e2_ref[...]

    best_idx = None
    run = None
    for c in range(N_CODES // CHUNK):
        mm2 = jax.lax.dot_general(
            xb, e2[:, c * CHUNK:(c + 1) * CHUNK],
            dimension_numbers=(((1,), (0,)), ((), ())),
            preferred_element_type=jnp.float32,
        )
        tc = (x_norm - mm2) + en[:, c * CHUNK:(c + 1) * CHUNK]
        m = jnp.min(tc, axis=1, keepdims=True)
        ids = ids_ref[:, c * CHUNK:(c + 1) * CHUNK]
        idx = jnp.min(jnp.where(tc == m, ids, 2.0 * N_CODES), axis=1)
        mval = m[:, 0]
        uv = jax.lax.bitcast_convert_type(mval, jnp.uint32)
        rv = (uv + jnp.uint32(0x7FFF) + ((uv >> 16) & jnp.uint32(1))) & jnp.uint32(0xFFFF0000)
        mval_b = jax.lax.bitcast_convert_type(rv, jnp.float32)
        if c == 0:
            best_idx = idx
            run = mval_b
        else:
            better = mval < run
            best_idx = jnp.where(better, idx, best_idx)
            run = jnp.where(better, mval_b, run)
    out_ref[0, 0, :] = best_idx.astype(jnp.int32)


@jax.jit
def kernel(x, embed):
    flat = x.reshape(-1, DIM)
    rows = flat.shape[0]
    nr = rows // BR
    embed2 = embed + embed
    embed_norm = jnp.sum(embed * embed, axis=0, keepdims=True)
    x_norm = jnp.sum(flat * flat, axis=1, keepdims=True).reshape(nr, 1, BR)
    out = pl.pallas_call(
        _assign_kernel,
        grid=(nr,),
        in_specs=[
            pl.BlockSpec((BR, DIM), lambda i: (i, 0)),
            pl.BlockSpec((1, 1, BR), lambda i: (i, 0, 0)),
            pl.BlockSpec((DIM, N_CODES), lambda i: (0, 0)),
            pl.BlockSpec((1, N_CODES), lambda i: (0, 0)),
            pl.BlockSpec((1, N_CODES), lambda i: (0, 0)),
        ],
        out_specs=pl.BlockSpec((1, 1, BR), lambda i: (i, 0, 0)),
        out_shape=jax.ShapeDtypeStruct((nr, 1, BR), jnp.int32),
    )(flat, x_norm, embed2, embed_norm,
      jnp.arange(N_CODES, dtype=jnp.float32).reshape(1, N_CODES))
    return out.reshape(x.shape[:-1])
